# baseline (speedup 1.0000x reference)
import math

import jax
import jax.numpy as jnp
from jax.experimental import pallas as pl
from jax.experimental.pallas import tpu as pltpu

_LOG_2PI = math.log(2.0 * math.pi)


def _ll_kernel(x_ref, ce_ref, out_ref, ta_ref, tb_ref, const_ref):
    d = x_ref.shape[1]

    @pl.when(pl.program_id(0) == 0)
    def _prep():
        ce = ce_ref[...]
        mean = ce[:, :d]
        log_sigma = ce[:, d:]
        e = jnp.exp(-2.0 * log_sigma)
        me = mean * e
        const = -0.5 * (
            d * _LOG_2PI
            + 2.0 * jnp.sum(log_sigma, axis=1)
            + jnp.sum(mean * me, axis=1)
        )
        ta_ref[...] = (-0.5 * (e - 1.0)).T.astype(jnp.bfloat16)
        tb_ref[...] = me.T.astype(jnp.bfloat16)
        const_ref[...] = const[None, :]

    x = x_ref[...]
    x2 = x * x
    rowsum = -0.5 * jnp.sum(x2, axis=1, keepdims=True)
    dn = (((1,), (0,)), ((), ()))
    acc = jax.lax.dot_general(
        x2.astype(jnp.bfloat16), ta_ref[...], dn,
        preferred_element_type=jnp.float32,
    ) + jax.lax.dot_general(
        x.astype(jnp.bfloat16), tb_ref[...], dn,
        preferred_element_type=jnp.float32,
    )
    out_ref[...] = acc + rowsum + const_ref[...]


def kernel(x, class_embed):
    b, d = x.shape
    c = class_embed.shape[0]
    tb = 256
    return pl.pallas_call(
        _ll_kernel,
        grid=(b // tb,),
        in_specs=[
            pl.BlockSpec((tb, d), lambda i: (i, 0)),
            pl.BlockSpec((c, 2 * d), lambda i: (0, 0)),
        ],
        out_specs=pl.BlockSpec((tb, c), lambda i: (i, 0)),
        out_shape=jax.ShapeDtypeStruct((b, c), jnp.float32),
        scratch_shapes=[
            pltpu.VMEM((d, c), jnp.bfloat16),
            pltpu.VMEM((d, c), jnp.bfloat16),
            pltpu.VMEM((1, c), jnp.float32),
        ],
        compiler_params=pltpu.CompilerParams(
            dimension_semantics=("arbitrary",),
        ),
    )(x, class_embed)


# speedup vs baseline: 1.0125x; 1.0125x over previous
import math

import jax
import jax.numpy as jnp
from jax.experimental import pallas as pl
from jax.experimental.pallas import tpu as pltpu

_LOG_2PI = math.log(2.0 * math.pi)


def _ll_kernel(x_ref, ce_hbm_ref, out_ref,
               ce_ref, ta_ref, tb_ref, const_ref, sem):
    d = x_ref.shape[1]

    @pl.when(pl.program_id(0) == 0)
    def _prep():
        copy = pltpu.make_async_copy(ce_hbm_ref, ce_ref, sem)
        copy.start()
        copy.wait()
        ce = ce_ref[...]
        mean = ce[:, :d]
        log_sigma = ce[:, d:]
        e = jnp.exp(-2.0 * log_sigma)
        me = mean * e
        const = -0.5 * (
            d * _LOG_2PI
            + 2.0 * jnp.sum(log_sigma, axis=1)
            + jnp.sum(mean * me, axis=1)
        )
        ta_ref[...] = (-0.5 * (e - 1.0)).T.astype(jnp.bfloat16)
        tb_ref[...] = me.T.astype(jnp.bfloat16)
        const_ref[...] = const[None, :]

    x = x_ref[...]
    x2 = x * x
    rowsum = -0.5 * jnp.sum(x2, axis=1, keepdims=True)
    dn = (((1,), (0,)), ((), ()))
    acc = jax.lax.dot_general(
        x2.astype(jnp.bfloat16), ta_ref[...], dn,
        preferred_element_type=jnp.float32,
    ) + jax.lax.dot_general(
        x.astype(jnp.bfloat16), tb_ref[...], dn,
        preferred_element_type=jnp.float32,
    )
    out_ref[...] = acc + rowsum + const_ref[...]


def kernel(x, class_embed):
    b, d = x.shape
    c = class_embed.shape[0]
    tb = 512
    return pl.pallas_call(
        _ll_kernel,
        grid=(b // tb,),
        in_specs=[
            pl.BlockSpec((tb, d), lambda i: (i, 0)),
            pl.BlockSpec(memory_space=pl.ANY),
        ],
        out_specs=pl.BlockSpec((tb, c), lambda i: (i, 0)),
        out_shape=jax.ShapeDtypeStruct((b, c), jnp.float32),
        scratch_shapes=[
            pltpu.VMEM((c, 2 * d), jnp.float32),
            pltpu.VMEM((d, c), jnp.bfloat16),
            pltpu.VMEM((d, c), jnp.bfloat16),
            pltpu.VMEM((1, c), jnp.float32),
            pltpu.SemaphoreType.DMA,
        ],
        compiler_params=pltpu.CompilerParams(
            dimension_semantics=("arbitrary",),
        ),
    )(x, class_embed)


# speedup vs baseline: 1.1112x; 1.0975x over previous
import math

import jax
import jax.numpy as jnp
from jax.experimental import pallas as pl
from jax.experimental.pallas import tpu as pltpu

_LOG_2PI = math.log(2.0 * math.pi)


def _ll_kernel(x_ref, ce_ref, out_ref, ta_ref, tb_ref, const_ref):
    d = x_ref.shape[1]

    @pl.when(pl.program_id(0) == 0)
    def _prep():
        ce = ce_ref[...]
        mean = ce[:, :d]
        log_sigma = ce[:, d:]
        e = jnp.exp(-2.0 * log_sigma)
        me = mean * e
        g = 2.0 * log_sigma + mean * me
        big = jnp.concatenate([-0.5 * (e - 1.0), me, g], axis=1).T
        ta_ref[...] = big[:d].astype(jnp.bfloat16)
        tb_ref[...] = big[d:2 * d].astype(jnp.bfloat16)
        const_ref[...] = -0.5 * (
            d * _LOG_2PI + jnp.sum(big[2 * d:], axis=0, keepdims=True)
        )

    x = x_ref[...]
    x2 = x * x
    rowsum = -0.5 * jnp.sum(x2, axis=1, keepdims=True)
    dn = (((1,), (0,)), ((), ()))
    acc = jax.lax.dot_general(
        x2.astype(jnp.bfloat16), ta_ref[...], dn,
        preferred_element_type=jnp.float32,
    ) + jax.lax.dot_general(
        x.astype(jnp.bfloat16), tb_ref[...], dn,
        preferred_element_type=jnp.float32,
    )
    out_ref[...] = acc + rowsum + const_ref[...]


def kernel(x, class_embed):
    b, d = x.shape
    c = class_embed.shape[0]
    tb = 512
    return pl.pallas_call(
        _ll_kernel,
        grid=(b // tb,),
        in_specs=[
            pl.BlockSpec((tb, d), lambda i: (i, 0)),
            pl.BlockSpec((c, 2 * d), lambda i: (0, 0)),
        ],
        out_specs=pl.BlockSpec((tb, c), lambda i: (i, 0)),
        out_shape=jax.ShapeDtypeStruct((b, c), jnp.float32),
        scratch_shapes=[
            pltpu.VMEM((d, c), jnp.bfloat16),
            pltpu.VMEM((d, c), jnp.bfloat16),
            pltpu.VMEM((1, c), jnp.float32),
        ],
        compiler_params=pltpu.CompilerParams(
            dimension_semantics=("arbitrary",),
        ),
    )(x, class_embed)
